# Initial kernel scaffold; baseline (speedup 1.0000x reference)
import jax
import jax.numpy as jnp
from jax.experimental import pallas as pl


def kernel(tokens, Wg, We, be, Wc):
    raise NotImplementedError("write your pallas kernel here")


# speedup vs baseline: 5.7114x; 5.7114x over previous
import functools

import jax
import jax.numpy as jnp
from jax import lax
from jax.experimental import pallas as pl
from jax.experimental.pallas import tpu as pltpu
from jax.experimental.pallas import tpu_sc as plsc

TOK = 8192
HID = 2048
E = 8

MBLK = 256
MBLKS = TOK // MBLK
NL = MBLKS + E - 1
NLP = 40

GATE_BLK = 512

NC = 2
NS = 16
NW = NC * NS
RPW = TOK // NW
CH = 32


def _gate_body(x_ref, wg_ref, top1_ref):
    x = x_ref[...]
    wg = wg_ref[...]
    logits = lax.dot_general(x, wg, (((1,), (1,)), ((), ())),
                             preferred_element_type=jnp.float32)
    mx = jnp.max(logits, axis=1, keepdims=True)
    eidx = lax.broadcasted_iota(jnp.int32, logits.shape, 1)
    cand = jnp.where(logits == mx, eidx, E)
    top1_ref[...] = jnp.min(cand, axis=1, keepdims=True)


def _gate(tokens, Wg):
    return pl.pallas_call(
        _gate_body,
        grid=(TOK // GATE_BLK,),
        in_specs=[
            pl.BlockSpec((GATE_BLK, HID), lambda i: (i, 0)),
            pl.BlockSpec((E, HID), lambda i: (0, 0)),
        ],
        out_specs=pl.BlockSpec((GATE_BLK, 1), lambda i: (i, 0)),
        out_shape=jax.ShapeDtypeStruct((TOK, 1), jnp.int32),
    )(tokens, Wg)


def _route_body(top1_ref, pos_ref, starts_ref, ends_ref, bid_ref, eid_ref,
                valid_ref):
    t1 = top1_ref[...]
    eids = lax.broadcasted_iota(jnp.int32, (TOK, E), 1)
    onehot = (t1 == eids).astype(jnp.int32)

    csum = onehot
    k = 1
    while k < TOK:
        csum = csum + jnp.concatenate(
            [jnp.zeros((k, E), jnp.int32), csum[:TOK - k, :]], axis=0)
        k *= 2
    counts = csum[TOK - 1:TOK, :]

    incl = counts
    k = 1
    while k < E:
        incl = incl + jnp.concatenate(
            [jnp.zeros((1, k), jnp.int32), incl[:, :E - k]], axis=1)
        k *= 2
    starts = incl - counts
    ends = incl

    rank_excl = csum - onehot
    pos = jnp.sum(onehot * (starts + rank_excl), axis=1, keepdims=True)
    pos_ref[...] = pos
    starts_ref[...] = starts
    ends_ref[...] = ends

    nonempty = counts > 0
    first_blk = starts // MBLK
    last_blk = (ends - 1) // MBLK
    tiles = jnp.where(nonempty, last_blk - first_blk + 1, 0)
    ct = tiles
    k = 1
    while k < E:
        ct = ct + jnp.concatenate(
            [jnp.zeros((1, k), jnp.int32), ct[:, :E - k]], axis=1)
        k *= 2
    ct_excl = ct - tiles
    total = ct[:, E - 1:E]

    l_ids = lax.broadcasted_iota(jnp.int32, (NLP, 1), 0)
    ct_b = jnp.broadcast_to(ct, (NLP, E))
    g = jnp.sum((ct_b <= l_ids).astype(jnp.int32), axis=1, keepdims=True)
    valid = l_ids < total
    gc = jnp.minimum(g, E - 1)
    onehot_g = (gc == lax.broadcasted_iota(jnp.int32, (NLP, E), 1)).astype(
        jnp.int32)
    fb = jnp.sum(onehot_g * jnp.broadcast_to(first_blk, (NLP, E)), axis=1,
                 keepdims=True)
    cte = jnp.sum(onehot_g * jnp.broadcast_to(ct_excl, (NLP, E)), axis=1,
                  keepdims=True)
    bid = fb + (l_ids - cte)
    bid_ref[...] = jnp.where(valid, bid, MBLKS - 1)
    eid_ref[...] = jnp.where(valid, gc, E - 1)
    valid_ref[...] = valid.astype(jnp.int32)


def _route(top1):
    return pl.pallas_call(
        _route_body,
        out_shape=(
            jax.ShapeDtypeStruct((TOK, 1), jnp.int32),
            jax.ShapeDtypeStruct((1, E), jnp.int32),
            jax.ShapeDtypeStruct((1, E), jnp.int32),
            jax.ShapeDtypeStruct((NLP, 1), jnp.int32),
            jax.ShapeDtypeStruct((NLP, 1), jnp.int32),
            jax.ShapeDtypeStruct((NLP, 1), jnp.int32),
        ),
    )(top1)


def _sc_mesh():
    return plsc.VectorSubcoreMesh(core_axis_name="c", subcore_axis_name="s")


def _sc_scatter_rows(src, pos):
    @functools.partial(
        pl.kernel,
        out_type=jax.ShapeDtypeStruct((TOK, HID), jnp.float32),
        mesh=_sc_mesh(),
        scratch_types=[
            pltpu.VMEM((CH,), jnp.int32),
            pltpu.VMEM((CH, HID), jnp.float32),
            pltpu.SemaphoreType.DMA,
        ],
    )
    def body(src_hbm, pos_hbm, out_hbm, idx_v, rows_v, sem):
        wid = lax.axis_index("s") * NC + lax.axis_index("c")
        base = wid * RPW

        def chunk(c, carry):
            off = base + c * CH
            pltpu.sync_copy(pos_hbm.at[pl.ds(off, CH)], idx_v)
            pltpu.sync_copy(src_hbm.at[pl.ds(off, CH)], rows_v)
            pltpu.async_copy(rows_v, out_hbm.at[idx_v], sem).wait()
            return carry

        lax.fori_loop(0, RPW // CH, chunk, 0)

    return body(src, pos)


def _sc_gather_rows(src, pos):
    @functools.partial(
        pl.kernel,
        out_type=jax.ShapeDtypeStruct((TOK, HID), jnp.float32),
        mesh=_sc_mesh(),
        scratch_types=[
            pltpu.VMEM((CH,), jnp.int32),
            pltpu.VMEM((CH, HID), jnp.float32),
            pltpu.SemaphoreType.DMA,
        ],
    )
    def body(src_hbm, pos_hbm, out_hbm, idx_v, rows_v, sem):
        wid = lax.axis_index("s") * NC + lax.axis_index("c")
        base = wid * RPW

        def chunk(c, carry):
            off = base + c * CH
            pltpu.sync_copy(pos_hbm.at[pl.ds(off, CH)], idx_v)
            pltpu.async_copy(src_hbm.at[idx_v], rows_v, sem).wait()
            pltpu.sync_copy(rows_v, out_hbm.at[pl.ds(off, CH)])
            return carry

        lax.fori_loop(0, RPW // CH, chunk, 0)

    return body(src, pos)


def _moe_body(bid_ref, eid_ref, valid_ref, s_ref, e_ref,
              xs_ref, we_ref, be_ref, wc_ref, out_ref):
    l = pl.program_id(0)

    @pl.when(valid_ref[l] == 1)
    def _():
        x = xs_ref[...]
        w = we_ref[0]
        h = lax.dot_general(x, w, (((1,), (1,)), ((), ())),
                            preferred_element_type=jnp.float32)
        h = h + be_ref[0]
        h = 0.5 * h * (1.0 + lax.erf(h * 0.7071067811865476))
        g = eid_ref[l]
        rows = bid_ref[l] * MBLK + lax.broadcasted_iota(
            jnp.int32, (MBLK, 1), 0)
        mask = (rows >= s_ref[g]) & (rows < e_ref[g])
        hm = jnp.where(mask, h, 0.0)
        y = lax.dot_general(hm, wc_ref[...], (((1,), (1,)), ((), ())),
                            preferred_element_type=jnp.float32)
        out_ref[...] = jnp.where(mask, y, out_ref[...])


def _moe(bid, eid, valid, starts, ends, xs, We, be, Wc):
    grid_spec = pltpu.PrefetchScalarGridSpec(
        num_scalar_prefetch=5,
        grid=(NL,),
        in_specs=[
            pl.BlockSpec((MBLK, HID), lambda l, b, e, v, s, en: (b[l], 0)),
            pl.BlockSpec((1, HID, HID), lambda l, b, e, v, s, en: (e[l], 0, 0)),
            pl.BlockSpec((1, 1, HID), lambda l, b, e, v, s, en: (e[l], 0, 0)),
            pl.BlockSpec((HID, HID), lambda l, b, e, v, s, en: (0, 0)),
        ],
        out_specs=pl.BlockSpec((MBLK, HID), lambda l, b, e, v, s, en: (b[l], 0)),
    )
    return pl.pallas_call(
        _moe_body,
        grid_spec=grid_spec,
        out_shape=jax.ShapeDtypeStruct((TOK, HID), jnp.float32),
        compiler_params=pltpu.CompilerParams(
            dimension_semantics=("arbitrary",)),
    )(bid, eid, valid, starts, ends, xs, We, be.reshape(E, 1, HID), Wc)


def kernel(tokens, Wg, We, be, Wc):
    top1 = _gate(tokens, Wg)
    pos2, starts2, ends2, bid2, eid2, valid2 = _route(top1)
    pos = pos2.reshape(TOK)
    bid = bid2.reshape(NLP)
    eid = eid2.reshape(NLP)
    valid = valid2.reshape(NLP)
    starts = starts2.reshape(E)
    ends = ends2.reshape(E)
    xs = _sc_scatter_rows(tokens, pos)
    ys = _moe(bid, eid, valid, starts, ends, xs, We, be, Wc)
    return _sc_gather_rows(ys, pos)
